# baseline (speedup 1.0000x reference)
import functools

import numpy as np

import jax
import jax.numpy as jnp
from jax import lax
from jax.experimental import pallas as pl
from jax.experimental.pallas import tpu as pltpu
from jax.experimental.pallas import tpu_sc as plsc

N = 10000
E = 320000
IN_DIM = 128
OUT_DIM = 64

NC = 2
NS = 16
NW = NC * NS
NPAD = 10240
CH = 64
NCHUNK = 160
EPW = NCHUNK * CH
EPAD = NW * EPW
NROWS_T = NPAD // NS

CHB = 128
NCHB = 160
EPT = NCHB * CHB

_HIGH = lax.Precision.HIGHEST

_mesh = plsc.VectorSubcoreMesh(
    core_axis_name="c", subcore_axis_name="s", num_cores=NC, num_subcores=NS
)
_sc_params = pltpu.CompilerParams(
    needs_layout_passes=False, use_tc_tiling_on_sc=False)

_PERM = np.empty(OUT_DIM, np.int64)
for _k in range(OUT_DIM // 32):
    for _i in range(16):
        _PERM[32 * _k + 2 * _i] = 32 * _k + _i
        _PERM[32 * _k + 2 * _i + 1] = 32 * _k + 16 + _i
_PMAT = np.zeros((OUT_DIM, OUT_DIM), np.float32)
_PMAT[_PERM, np.arange(OUT_DIM)] = 1.0


def _prep_body(x_ref, wgcn_ref, wg_ref, wgat_ref, wa_ref, att2_ref,
               g_ref, asad_ref):
    x = x_ref[...]
    m1 = jnp.dot(wgcn_ref[...], wg_ref[...],
                 preferred_element_type=jnp.float32, precision=_HIGH)
    m2 = jnp.dot(wgat_ref[...], wa_ref[...],
                 preferred_element_type=jnp.float32, precision=_HIGH)
    g1 = jnp.dot(x, m1, preferred_element_type=jnp.float32, precision=_HIGH)
    g2 = jnp.dot(x, m2, preferred_element_type=jnp.float32, precision=_HIGH)
    g_ref[...] = jnp.concatenate([g1, g2], axis=1)
    u = jnp.dot(wgat_ref[...], att2_ref[...],
                preferred_element_type=jnp.float32, precision=_HIGH)
    asad_ref[...] = lax.dot_general(
        u, x, (((0,), (1,)), ((), ())),
        preferred_element_type=jnp.float32, precision=_HIGH)


def _prep(xpad, W_gcn, w_gcn, W_gat, w_gat, att2):
    return pl.pallas_call(
        _prep_body,
        out_shape=(
            jax.ShapeDtypeStruct((NPAD, IN_DIM), jnp.float32),
            jax.ShapeDtypeStruct((2, NPAD), jnp.float32),
        ),
    )(xpad, W_gcn, w_gcn, W_gat, w_gat, att2)


@functools.partial(
    pl.kernel,
    out_type=(
        jax.ShapeDtypeStruct((NW, EPW), jnp.float32),
        jax.ShapeDtypeStruct((NC, NPAD), jnp.float32),
        jax.ShapeDtypeStruct((NC, NPAD), jnp.float32),
    ),
    mesh=_mesh,
    scratch_types=(
        pltpu.VMEM((NPAD,), jnp.float32),
        pltpu.VMEM((NPAD,), jnp.float32),
        pltpu.VMEM((NCHUNK, CH), jnp.int32),
        pltpu.VMEM((NCHUNK, CH), jnp.int32),
        pltpu.VMEM((EPW,), jnp.float32),
        pltpu.VMEM((CH,), jnp.float32),
        pltpu.VMEM((NROWS_T,), jnp.float32),
        pltpu.VMEM_SHARED((NPAD,), jnp.float32),
        pltpu.VMEM_SHARED((NPAD,), jnp.float32),
        pltpu.SemaphoreType.DMA,
        pltpu.SemaphoreType.DMA,
    ),
    compiler_params=_sc_params,
)
def _edge_scalar_kernel(srcp, dstp, asad, ee_out, degp, denp,
                        as_v, ad_v, src_v, dst_v, ee_v, ones_v, zero_v,
                        deg_sh, den_sh, sem1, sem2):
    c = lax.axis_index("c")
    s = lax.axis_index("s")
    w = s * NC + c
    zf = jnp.zeros((16,), jnp.float32)
    of = jnp.ones((16,), jnp.float32)

    pltpu.sync_copy(asad.at[0], as_v)
    pltpu.sync_copy(asad.at[1], ad_v)
    pltpu.sync_copy(srcp.at[w], src_v)
    pltpu.sync_copy(dstp.at[w], dst_v)

    @pl.loop(0, CH // 16)
    def _fill_ones(i):
        ones_v[pl.ds(i * 16, 16)] = of

    @pl.loop(0, NROWS_T // 16)
    def _fill_zeros(i):
        zero_v[pl.ds(i * 16, 16)] = zf

    pltpu.sync_copy(zero_v, deg_sh.at[pl.ds(s * NROWS_T, NROWS_T)])
    pltpu.sync_copy(zero_v, den_sh.at[pl.ds(s * NROWS_T, NROWS_T)])
    plsc.subcore_barrier()

    @pl.loop(0, NCHUNK)
    def _chunk(j):
        @pl.loop(0, CH // 16)
        def _grp(g):
            sl = pl.ds(g * 16, 16)
            s16 = src_v[j, sl]
            d16 = dst_v[j, sl]
            asv = plsc.load_gather(as_v, [s16])
            adv = plsc.load_gather(ad_v, [d16])
            e = asv + adv
            e = jnp.maximum(e, 0.2 * e)
            ee = jnp.exp(e)
            ee_v[pl.ds(j * CH + g * 16, 16)] = ee

        cp1 = pltpu.async_copy(ones_v, deg_sh.at[dst_v.at[j]], sem1, add=True)
        cp2 = pltpu.async_copy(ee_v.at[pl.ds(j * CH, CH)],
                               den_sh.at[dst_v.at[j]], sem2, add=True)
        cp1.wait()
        cp2.wait()

    pltpu.sync_copy(ee_v, ee_out.at[w])
    plsc.subcore_barrier()

    pltpu.sync_copy(deg_sh.at[pl.ds(s * NROWS_T, NROWS_T)],
                    degp.at[c, pl.ds(s * NROWS_T, NROWS_T)])
    pltpu.sync_copy(den_sh.at[pl.ds(s * NROWS_T, NROWS_T)],
                    denp.at[c, pl.ds(s * NROWS_T, NROWS_T)])


def _mid_body(degp_ref, denp_ref, asad_ref, g_ref, pmat_ref,
              vec_ref, gp0_ref, gp1_ref):
    deg = degp_ref[0] + degp_ref[1] + 1.0
    e_self = asad_ref[0] + asad_ref[1]
    e_self = jnp.maximum(e_self, 0.2 * e_self)
    ee_self = jnp.exp(e_self)
    den = denp_ref[0] + denp_ref[1] + ee_self
    dinv = lax.rsqrt(deg)
    rden = 1.0 / den
    invdeg = 1.0 / deg
    selfgat = ee_self * rden
    vec_ref[...] = jnp.stack([dinv, rden, invdeg, selfgat], axis=0)
    g = g_ref[...]
    pmat = pmat_ref[...]
    gp0_ref[...] = jnp.dot(g[:, :OUT_DIM] * dinv[:, None], pmat,
                           preferred_element_type=jnp.float32,
                           precision=_HIGH).astype(jnp.bfloat16)
    gp1_ref[...] = jnp.dot(g[:, OUT_DIM:], pmat,
                           preferred_element_type=jnp.float32,
                           precision=_HIGH).astype(jnp.bfloat16)


def _mid(degp, denp, asad, G, pmat):
    return pl.pallas_call(
        _mid_body,
        out_shape=(
            jax.ShapeDtypeStruct((4, NPAD), jnp.float32),
            jax.ShapeDtypeStruct((NPAD, OUT_DIM), jnp.bfloat16),
            jax.ShapeDtypeStruct((NPAD, OUT_DIM), jnp.bfloat16),
        ),
    )(degp, denp, asad, G, pmat)


@functools.partial(
    pl.kernel,
    out_type=jax.ShapeDtypeStruct((NC, NPAD, OUT_DIM), jnp.float32),
    mesh=_mesh,
    scratch_types=(
        pltpu.VMEM((NCHB, CHB), jnp.int32),
        pltpu.VMEM((NCHB, CHB), jnp.int32),
        pltpu.VMEM((CHB,), jnp.float32),
        pltpu.VMEM((CHB,), jnp.float32),
        pltpu.VMEM((CHB, OUT_DIM), jnp.bfloat16),
        pltpu.VMEM((CHB, OUT_DIM), jnp.bfloat16),
        pltpu.VMEM((CHB, OUT_DIM), jnp.float32),
        pltpu.VMEM((CHB, OUT_DIM), jnp.float32),
        pltpu.VMEM_SHARED((NPAD, OUT_DIM), jnp.float32),
        pltpu.SemaphoreType.DMA,
        pltpu.SemaphoreType.DMA,
        pltpu.SemaphoreType.DMA,
        pltpu.SemaphoreType.DMA,
    ),
    compiler_params=_sc_params,
)
def _edge_row_kernel(srcp, dstp, eep, gp0, gp1, accp,
                     src_v, dst_v, ee0, ee1, rbf0, rbf1, rf0, rf1,
                     acc_sh, gs0, gs1, ss0, ss1):
    c = lax.axis_index("c")
    s = lax.axis_index("s")
    pltpu.sync_copy(srcp.at[s], src_v)
    pltpu.sync_copy(dstp.at[s], dst_v)

    rsl = pl.ds(s * NROWS_T, NROWS_T)
    zf = jnp.zeros((16,), jnp.float32)

    @pl.loop(0, CHB)
    def _zero(r):
        for k in range(OUT_DIM // 16):
            rf1[r, pl.ds(k * 16, 16)] = zf

    for i in range(NROWS_T // CHB):
        pltpu.sync_copy(rf1, acc_sh.at[pl.ds(s * NROWS_T + i * CHB, CHB)])
    plsc.subcore_barrier()

    bf_bufs = (rbf0, rbf1)
    f_bufs = (rf0, rf1)
    ee_bufs = (ee0, ee1)
    gsems = (gs0, gs1)
    ssems = (ss0, ss1)

    def _main_loop(gp, with_scale):
        pltpu.async_copy(gp.at[src_v.at[0]], rbf0, gs0)
        if with_scale:
            pltpu.async_copy(eep.at[s, pl.ds(0, CHB)], ee0, gs0)

        @pl.loop(0, NCHB // 2)
        def _pair(t):
            for b in range(2):
                jj = t * 2 + b
                rbf, rf, eb = bf_bufs[b], f_bufs[b], ee_bufs[b]
                gb, sb = gsems[b], ssems[b]
                rbf_o, eb_o = bf_bufs[1 - b], ee_bufs[1 - b]
                rf_o = f_bufs[1 - b]
                go, so = gsems[1 - b], ssems[1 - b]

                pltpu.make_async_copy(gp.at[src_v.at[jj]], rbf, gb).wait()
                if with_scale:
                    pltpu.make_async_copy(eep.at[s, pl.ds(jj * CHB, CHB)],
                                          eb, gb).wait()

                @pl.when(jnp.logical_and(jj >= 1, jj + 1 < NCHB))
                def _drain():
                    pltpu.make_async_copy(
                        rf_o, acc_sh.at[dst_v.at[jj - 1]], so).wait()

                @pl.when(jj + 1 < NCHB)
                def _prefetch():
                    pltpu.async_copy(gp.at[src_v.at[jj + 1]], rbf_o, go)
                    if with_scale:
                        pltpu.async_copy(
                            eep.at[s, pl.ds((jj + 1) * CHB, CHB)], eb_o, go)

                @pl.loop(0, CHB, unroll=2)
                def _cv(e):
                    if with_scale:
                        eev = plsc.load_gather(
                            eb, [jnp.full((16,), e, jnp.int32)])
                    for k in range(OUT_DIM // 32):
                        v = rbf[e, pl.ds(k * 32, 32)]
                        va, vb = plsc.unpack(
                            v, format=plsc.PackFormat.INTERLEAVED)
                        if with_scale:
                            va = va * eev
                            vb = vb * eev
                        rf[e, pl.ds(k * 32, 16)] = va
                        rf[e, pl.ds(k * 32 + 16, 16)] = vb

                pltpu.async_copy(rf, acc_sh.at[dst_v.at[jj]], sb, add=True)

        pltpu.make_async_copy(rf0, acc_sh.at[dst_v.at[NCHB - 2]],
                              ss0).wait()
        pltpu.make_async_copy(rf1, acc_sh.at[dst_v.at[NCHB - 1]],
                              ss1).wait()

    @pl.when(c == 0)
    def _gcn():
        _main_loop(gp0, False)

    @pl.when(c == 1)
    def _gat():
        _main_loop(gp1, True)

    plsc.subcore_barrier()
    pltpu.sync_copy(acc_sh.at[rsl], accp.at[c, rsl])


def _final_body(accp_ref, g_ref, vec_ref, b2_ref, wg_ref, wa_ref, out_ref):
    acc_gcn = accp_ref[0]
    acc_gat = accp_ref[1]
    g = g_ref[...]
    dinv = vec_ref[0]
    rden = vec_ref[1]
    invdeg = vec_ref[2]
    selfgat = vec_ref[3]
    y = (acc_gcn * dinv[:, None]
         + acc_gat * rden[:, None]
         + g[:, :OUT_DIM] * invdeg[:, None]
         + g[:, OUT_DIM:] * selfgat[:, None])
    bias = (jnp.dot(b2_ref[0:1], wg_ref[...],
                    preferred_element_type=jnp.float32, precision=_HIGH)
            + jnp.dot(b2_ref[1:2], wa_ref[...],
                      preferred_element_type=jnp.float32, precision=_HIGH))
    out_ref[...] = jnp.tanh(y + bias)[:N]


def _final(accp, G, vec, b2, w_gcn, w_gat):
    return pl.pallas_call(
        _final_body,
        out_shape=jax.ShapeDtypeStruct((N, OUT_DIM), jnp.float32),
    )(accp, G, vec, b2, w_gcn, w_gat)


def kernel(x, edge_index, W_gcn, b_gcn, W_gat, att_src, att_dst, b_gat,
           w_gcn, w_gat):
    xpad = jnp.pad(x, ((0, NPAD - N), (0, 0)))
    pad_idx = jnp.full((EPAD - E,), N, jnp.int32)
    srcp = jnp.concatenate([edge_index[0], pad_idx]).reshape(NW, NCHUNK, CH)
    dstp = jnp.concatenate([edge_index[1], pad_idx]).reshape(NW, NCHUNK, CH)
    att2 = jnp.stack([att_src, att_dst], axis=1)
    b2 = jnp.stack([b_gcn, b_gat], axis=0)

    srcp_b = srcp.reshape(NS, NCHB, CHB)
    dstp_b = dstp.reshape(NS, NCHB, CHB)

    G, asad = _prep(xpad, W_gcn, w_gcn, W_gat, w_gat, att2)
    ee, degp, denp = _edge_scalar_kernel(srcp, dstp, asad)
    vec, gp0, gp1 = _mid(degp, denp, asad, G, jnp.asarray(_PMAT))
    accp = _edge_row_kernel(srcp_b, dstp_b, ee.reshape(NS, EPT), gp0, gp1)
    return _final(accp, G, vec, b2, w_gcn, w_gat)


# speedup vs baseline: 1.0502x; 1.0502x over previous
import functools

import numpy as np

import jax
import jax.numpy as jnp
from jax import lax
from jax.experimental import pallas as pl
from jax.experimental.pallas import tpu as pltpu
from jax.experimental.pallas import tpu_sc as plsc

N = 10000
E = 320000
IN_DIM = 128
OUT_DIM = 64

NC = 2
NS = 16
NW = NC * NS
NPAD = 10240
CH = 64
NCHUNK = 160
EPW = NCHUNK * CH
EPAD = NW * EPW
NROWS_T = NPAD // NS

CHB = 64
NCHB = 160

_HIGH = lax.Precision.HIGHEST

_mesh = plsc.VectorSubcoreMesh(
    core_axis_name="c", subcore_axis_name="s", num_cores=NC, num_subcores=NS
)
_sc_params = pltpu.CompilerParams(
    needs_layout_passes=False, use_tc_tiling_on_sc=False)

_PERM = np.empty(IN_DIM, np.int64)
for _k in range(IN_DIM // 32):
    for _i in range(16):
        _PERM[32 * _k + 2 * _i] = 32 * _k + _i
        _PERM[32 * _k + 2 * _i + 1] = 32 * _k + 16 + _i
_PMAT = np.zeros((IN_DIM, IN_DIM), np.float32)
_PMAT[_PERM, np.arange(IN_DIM)] = 1.0


def _prep_body(x_ref, wgcn_ref, wg_ref, wgat_ref, wa_ref, att2_ref,
               g_ref, asad_ref):
    x = x_ref[...]
    m1 = jnp.dot(wgcn_ref[...], wg_ref[...],
                 preferred_element_type=jnp.float32, precision=_HIGH)
    m2 = jnp.dot(wgat_ref[...], wa_ref[...],
                 preferred_element_type=jnp.float32, precision=_HIGH)
    g1 = jnp.dot(x, m1, preferred_element_type=jnp.float32, precision=_HIGH)
    g2 = jnp.dot(x, m2, preferred_element_type=jnp.float32, precision=_HIGH)
    g_ref[...] = jnp.concatenate([g1, g2], axis=1)
    u = jnp.dot(wgat_ref[...], att2_ref[...],
                preferred_element_type=jnp.float32, precision=_HIGH)
    asad_ref[...] = lax.dot_general(
        u, x, (((0,), (1,)), ((), ())),
        preferred_element_type=jnp.float32, precision=_HIGH)


def _prep(xpad, W_gcn, w_gcn, W_gat, w_gat, att2):
    return pl.pallas_call(
        _prep_body,
        out_shape=(
            jax.ShapeDtypeStruct((NPAD, IN_DIM), jnp.float32),
            jax.ShapeDtypeStruct((2, NPAD), jnp.float32),
        ),
    )(xpad, W_gcn, w_gcn, W_gat, w_gat, att2)


@functools.partial(
    pl.kernel,
    out_type=(
        jax.ShapeDtypeStruct((NW, EPW), jnp.float32),
        jax.ShapeDtypeStruct((NC, NPAD), jnp.float32),
        jax.ShapeDtypeStruct((NC, NPAD), jnp.float32),
    ),
    mesh=_mesh,
    scratch_types=(
        pltpu.VMEM((NPAD,), jnp.float32),
        pltpu.VMEM((NPAD,), jnp.float32),
        pltpu.VMEM((NCHUNK, CH), jnp.int32),
        pltpu.VMEM((NCHUNK, CH), jnp.int32),
        pltpu.VMEM((EPW,), jnp.float32),
        pltpu.VMEM((CH,), jnp.float32),
        pltpu.VMEM((NROWS_T,), jnp.float32),
        pltpu.VMEM_SHARED((NPAD,), jnp.float32),
        pltpu.VMEM_SHARED((NPAD,), jnp.float32),
        pltpu.SemaphoreType.DMA,
        pltpu.SemaphoreType.DMA,
    ),
    compiler_params=_sc_params,
)
def _edge_scalar_kernel(srcp, dstp, asad, ee_out, degp, denp,
                        as_v, ad_v, src_v, dst_v, ee_v, ones_v, zero_v,
                        deg_sh, den_sh, sem1, sem2):
    c = lax.axis_index("c")
    s = lax.axis_index("s")
    w = s * NC + c
    zf = jnp.zeros((16,), jnp.float32)
    of = jnp.ones((16,), jnp.float32)

    pltpu.sync_copy(asad.at[0], as_v)
    pltpu.sync_copy(asad.at[1], ad_v)
    pltpu.sync_copy(srcp.at[w], src_v)
    pltpu.sync_copy(dstp.at[w], dst_v)

    @pl.loop(0, CH // 16)
    def _fill_ones(i):
        ones_v[pl.ds(i * 16, 16)] = of

    @pl.loop(0, NROWS_T // 16)
    def _fill_zeros(i):
        zero_v[pl.ds(i * 16, 16)] = zf

    pltpu.sync_copy(zero_v, deg_sh.at[pl.ds(s * NROWS_T, NROWS_T)])
    pltpu.sync_copy(zero_v, den_sh.at[pl.ds(s * NROWS_T, NROWS_T)])
    plsc.subcore_barrier()

    @pl.loop(0, NCHUNK)
    def _chunk(j):
        @pl.loop(0, CH // 16)
        def _grp(g):
            sl = pl.ds(g * 16, 16)
            s16 = src_v[j, sl]
            d16 = dst_v[j, sl]
            asv = plsc.load_gather(as_v, [s16])
            adv = plsc.load_gather(ad_v, [d16])
            e = asv + adv
            e = jnp.maximum(e, 0.2 * e)
            ee = jnp.exp(e)
            ee_v[pl.ds(j * CH + g * 16, 16)] = ee

        cp1 = pltpu.async_copy(ones_v, deg_sh.at[dst_v.at[j]], sem1, add=True)
        cp2 = pltpu.async_copy(ee_v.at[pl.ds(j * CH, CH)],
                               den_sh.at[dst_v.at[j]], sem2, add=True)
        cp1.wait()
        cp2.wait()

    pltpu.sync_copy(ee_v, ee_out.at[w])
    plsc.subcore_barrier()

    pltpu.sync_copy(deg_sh.at[pl.ds(s * NROWS_T, NROWS_T)],
                    degp.at[c, pl.ds(s * NROWS_T, NROWS_T)])
    pltpu.sync_copy(den_sh.at[pl.ds(s * NROWS_T, NROWS_T)],
                    denp.at[c, pl.ds(s * NROWS_T, NROWS_T)])


def _mid_body(degp_ref, denp_ref, asad_ref, g_ref, pmat_ref,
              vec_ref, gp_ref):
    deg = degp_ref[0] + degp_ref[1] + 1.0
    e_self = asad_ref[0] + asad_ref[1]
    e_self = jnp.maximum(e_self, 0.2 * e_self)
    ee_self = jnp.exp(e_self)
    den = denp_ref[0] + denp_ref[1] + ee_self
    dinv = lax.rsqrt(deg)
    rden = 1.0 / den
    invdeg = 1.0 / deg
    selfgat = ee_self * rden
    vec_ref[...] = jnp.stack([dinv, rden, invdeg, selfgat], axis=0)
    g = g_ref[...]
    gp = jnp.concatenate(
        [g[:, :OUT_DIM] * dinv[:, None], g[:, OUT_DIM:]], axis=1)
    gp_ref[...] = jnp.dot(gp, pmat_ref[...],
                          preferred_element_type=jnp.float32,
                          precision=_HIGH).astype(jnp.bfloat16)


def _mid(degp, denp, asad, G, pmat):
    return pl.pallas_call(
        _mid_body,
        out_shape=(
            jax.ShapeDtypeStruct((4, NPAD), jnp.float32),
            jax.ShapeDtypeStruct((NPAD, IN_DIM), jnp.bfloat16),
        ),
    )(degp, denp, asad, G, pmat)


@functools.partial(
    pl.kernel,
    out_type=jax.ShapeDtypeStruct((NC, NPAD, IN_DIM), jnp.float32),
    mesh=_mesh,
    scratch_types=(
        pltpu.VMEM((NCHB, CHB), jnp.int32),
        pltpu.VMEM((NCHB, CHB), jnp.int32),
        pltpu.VMEM((CHB,), jnp.float32),
        pltpu.VMEM((CHB,), jnp.float32),
        pltpu.VMEM((CHB, IN_DIM), jnp.bfloat16),
        pltpu.VMEM((CHB, IN_DIM), jnp.bfloat16),
        pltpu.VMEM((CHB, IN_DIM), jnp.float32),
        pltpu.VMEM((CHB, IN_DIM), jnp.float32),
        pltpu.VMEM_SHARED((NPAD, IN_DIM), jnp.float32),
        pltpu.SemaphoreType.DMA,
        pltpu.SemaphoreType.DMA,
        pltpu.SemaphoreType.DMA,
        pltpu.SemaphoreType.DMA,
    ),
    compiler_params=_sc_params,
)
def _edge_row_kernel(srcp, dstp, eep, gp, accp,
                     src_v, dst_v, ee0, ee1, rbf0, rbf1, rf0, rf1,
                     acc_sh, gs0, gs1, ss0, ss1):
    c = lax.axis_index("c")
    s = lax.axis_index("s")
    w = s * NC + c
    pltpu.sync_copy(srcp.at[w], src_v)
    pltpu.sync_copy(dstp.at[w], dst_v)

    rsl = pl.ds(s * NROWS_T, NROWS_T)
    zf = jnp.zeros((16,), jnp.float32)

    pltpu.async_copy(gp.at[src_v.at[0]], rbf0, gs0)
    pltpu.async_copy(eep.at[w, pl.ds(0, CHB)], ee0, gs0)

    @pl.loop(0, CHB)
    def _zero(r):
        for k in range(IN_DIM // 16):
            rf1[r, pl.ds(k * 16, 16)] = zf

    for i in range(NROWS_T // CHB):
        pltpu.sync_copy(rf1, acc_sh.at[pl.ds(s * NROWS_T + i * CHB, CHB)])
    plsc.subcore_barrier()

    bf_bufs = (rbf0, rbf1)
    f_bufs = (rf0, rf1)
    ee_bufs = (ee0, ee1)
    gsems = (gs0, gs1)
    ssems = (ss0, ss1)

    @pl.loop(0, NCHB // 2)
    def _pair(t):
        for b in range(2):
            jj = t * 2 + b
            rbf, rf, eb = bf_bufs[b], f_bufs[b], ee_bufs[b]
            gb, sb = gsems[b], ssems[b]
            rbf_o, eb_o = bf_bufs[1 - b], ee_bufs[1 - b]
            rf_o = f_bufs[1 - b]
            go, so = gsems[1 - b], ssems[1 - b]

            pltpu.make_async_copy(gp.at[src_v.at[jj]], rbf, gb).wait()
            pltpu.make_async_copy(eep.at[w, pl.ds(jj * CHB, CHB)], eb,
                                  gb).wait()

            @pl.when(jnp.logical_and(jj >= 1, jj + 1 < NCHB))
            def _drain():
                pltpu.make_async_copy(
                    rf_o, acc_sh.at[dst_v.at[jj - 1]], so).wait()

            @pl.when(jj + 1 < NCHB)
            def _prefetch():
                pltpu.async_copy(gp.at[src_v.at[jj + 1]], rbf_o, go)
                pltpu.async_copy(eep.at[w, pl.ds((jj + 1) * CHB, CHB)],
                                 eb_o, go)

            @pl.loop(0, CHB, unroll=2)
            def _scale(e):
                eev = plsc.load_gather(eb, [jnp.full((16,), e, jnp.int32)])
                for k in range(IN_DIM // 32):
                    v = rbf[e, pl.ds(k * 32, 32)]
                    va, vb = plsc.unpack(
                        v, format=plsc.PackFormat.INTERLEAVED)
                    if k >= OUT_DIM // 32:
                        va = va * eev
                        vb = vb * eev
                    rf[e, pl.ds(k * 32, 16)] = va
                    rf[e, pl.ds(k * 32 + 16, 16)] = vb

            pltpu.async_copy(rf, acc_sh.at[dst_v.at[jj]], sb, add=True)

    pltpu.make_async_copy(rf0, acc_sh.at[dst_v.at[NCHB - 2]], ss0).wait()
    pltpu.make_async_copy(rf1, acc_sh.at[dst_v.at[NCHB - 1]], ss1).wait()
    plsc.subcore_barrier()
    pltpu.sync_copy(acc_sh.at[rsl], accp.at[c, rsl])


def _final_body(accp_ref, g_ref, vec_ref, b2_ref, wg_ref, wa_ref, out_ref):
    acc = accp_ref[0] + accp_ref[1]
    g = g_ref[...]
    dinv = vec_ref[0]
    rden = vec_ref[1]
    invdeg = vec_ref[2]
    selfgat = vec_ref[3]
    y = (acc[:, :OUT_DIM] * dinv[:, None]
         + acc[:, OUT_DIM:] * rden[:, None]
         + g[:, :OUT_DIM] * invdeg[:, None]
         + g[:, OUT_DIM:] * selfgat[:, None])
    bias = (jnp.dot(b2_ref[0:1], wg_ref[...],
                    preferred_element_type=jnp.float32, precision=_HIGH)
            + jnp.dot(b2_ref[1:2], wa_ref[...],
                      preferred_element_type=jnp.float32, precision=_HIGH))
    out_ref[...] = jnp.tanh(y + bias)[:N]


def _final(accp, G, vec, b2, w_gcn, w_gat):
    return pl.pallas_call(
        _final_body,
        out_shape=jax.ShapeDtypeStruct((N, OUT_DIM), jnp.float32),
    )(accp, G, vec, b2, w_gcn, w_gat)


def kernel(x, edge_index, W_gcn, b_gcn, W_gat, att_src, att_dst, b_gat,
           w_gcn, w_gat):
    xpad = jnp.pad(x, ((0, NPAD - N), (0, 0)))
    pad_idx = jnp.full((EPAD - E,), N, jnp.int32)
    srcp = jnp.concatenate([edge_index[0], pad_idx]).reshape(NW, NCHUNK, CH)
    dstp = jnp.concatenate([edge_index[1], pad_idx]).reshape(NW, NCHUNK, CH)
    att2 = jnp.stack([att_src, att_dst], axis=1)
    b2 = jnp.stack([b_gcn, b_gat], axis=0)

    srcp_b = srcp.reshape(NW, NCHB, CHB)
    dstp_b = dstp.reshape(NW, NCHB, CHB)

    G, asad = _prep(xpad, W_gcn, w_gcn, W_gat, w_gat, att2)
    ee, degp, denp = _edge_scalar_kernel(srcp, dstp, asad)
    vec, gp = _mid(degp, denp, asad, G, jnp.asarray(_PMAT))
    accp = _edge_row_kernel(srcp_b, dstp_b, ee, gp)
    return _final(accp, G, vec, b2, w_gcn, w_gat)
